# baseline (speedup 1.0000x reference)
import jax
import jax.numpy as jnp
from jax.experimental import pallas as pl
from jax.experimental.pallas import tpu as pltpu

_KMAX = 50
_KMIN = 50
_SIGN_MASK = 0x7FFFFFFF
_INT_MIN = -2147483648
_INT_MAX = 2147483647


def _select_body(x_ref, ones_ref, o_ref, keys_ref):
    rows, n = x_ref.shape
    bits = jax.lax.bitcast_convert_type(x_ref[...], jnp.int32)
    keys_ref[...] = jnp.where(bits >= 0, bits, bits ^ _SIGN_MASK)

    def counts(t, u):
        k = keys_ref[...]
        m1 = (k >= t).astype(jnp.float32)
        m2 = (k <= u).astype(jnp.float32)
        m = jnp.concatenate([m1, m2], axis=0)
        s = jax.lax.dot_general(ones_ref[...], m,
                                (((1,), (1,)), ((), ())),
                                preferred_element_type=jnp.float32)
        sT = s.reshape(2 * rows, 1)
        ct = sT[:rows].astype(jnp.int32)
        cu = sT[rows:].astype(jnp.int32)
        return ct, cu

    zero = jnp.zeros((rows, 1), jnp.int32)
    ct0, cu0 = counts(zero, zero - 1)
    t0 = jnp.where(ct0 >= _KMAX, zero, zero + _INT_MIN)
    u0 = jnp.where(cu0 >= _KMIN, zero - 1, zero + _INT_MAX)

    def body(i, carry):
        t, u = carry
        p = (1073741824 >> i).astype(jnp.int32)
        tt = t + p
        uu = u - p
        ct, cu = counts(tt, uu)
        return (jnp.where(ct >= _KMAX, tt, t), jnp.where(cu >= _KMIN, uu, u))

    t, u = jax.lax.fori_loop(0, 31, body, (t0, u0))

    cnt_gt, cnt_lt = counts(t + 1, u - 1)
    seg = n // 8
    gs, ls = [], []
    for sidx in range(8):
        ks = keys_ref[:, sidx * seg:(sidx + 1) * seg]
        xs = x_ref[:, sidx * seg:(sidx + 1) * seg]
        gs.append(jnp.sum(jnp.where(ks > t, xs, 0.0), axis=1, keepdims=True))
        ls.append(jnp.sum(jnp.where(ks < u, xs, 0.0), axis=1, keepdims=True))
    s_gt = sum(gs)
    s_lt = sum(ls)
    tval = jax.lax.bitcast_convert_type(
        jnp.where(t >= 0, t, t ^ _SIGN_MASK), jnp.float32)
    uval = jax.lax.bitcast_convert_type(
        jnp.where(u >= 0, u, u ^ _SIGN_MASK), jnp.float32)
    top = s_gt + (_KMAX - cnt_gt).astype(jnp.float32) * tval
    bot = s_lt + (_KMIN - cnt_lt).astype(jnp.float32) * uval
    res = top / _KMAX + bot / _KMIN
    o_ref[0] = jnp.broadcast_to(res, (rows, 128))


def kernel(inputs):
    b, h, w, c = inputs.shape
    n = h * w
    rows = b * c
    rg = 16
    assert rows % rg == 0 and n % 128 == 0
    g = rows // rg
    x = jnp.transpose(inputs, (0, 3, 1, 2)).reshape(rows, n)
    ones = jnp.ones((1, n), jnp.float32)
    out = pl.pallas_call(
        _select_body,
        grid=(g,),
        in_specs=[pl.BlockSpec((rg, n), lambda i: (i, 0)),
                  pl.BlockSpec((1, n), lambda i: (0, 0))],
        out_specs=pl.BlockSpec((1, rg, 128), lambda i: (i, 0, 0)),
        out_shape=jax.ShapeDtypeStruct((g, rg, 128), jnp.float32),
        scratch_shapes=[pltpu.VMEM((rg, n), jnp.int32)],
    )(x, ones)
    return out[:, :, 0].reshape(b, c)


# speedup vs baseline: 1.5270x; 1.5270x over previous
import jax
import jax.numpy as jnp
from jax.experimental import pallas as pl
from jax.experimental.pallas import tpu as pltpu

_KMAX = 50
_KMIN = 50
_SIGN_MASK = 0x7FFFFFFF
_INT_MIN = -2147483648
_INT_MAX = 2147483647


def _select_body(x_ref, o_ref, keys_ref):
    rows, n = x_ref.shape
    bits = jax.lax.bitcast_convert_type(x_ref[...], jnp.int32)
    keys_ref[...] = jnp.where(bits >= 0, bits, bits ^ _SIGN_MASK)

    nsplit = 8
    cseg = n // nsplit

    def counts(t, u):
        cts, cus = [], []
        for s in range(nsplit):
            k = keys_ref[:, s * cseg:(s + 1) * cseg]
            cts.append(jnp.sum((k >= t).astype(jnp.int32), axis=1,
                               keepdims=True))
            cus.append(jnp.sum((k <= u).astype(jnp.int32), axis=1,
                               keepdims=True))
        return sum(cts), sum(cus)

    def search_body(nbits):
        top = 1 << (nbits - 1)

        def body(i, carry):
            t, u = carry
            p = (top >> i).astype(jnp.int32)
            tt = t + p
            uu = u - p
            ct, cu = counts(tt, uu)
            return (jnp.where(ct >= _KMAX, tt, t),
                    jnp.where(cu >= _KMIN, uu, u))

        return body

    maxs, mins = [], []
    for s in range(nsplit):
        k = keys_ref[:, s * cseg:(s + 1) * cseg]
        maxs.append(jnp.max(k, axis=1, keepdims=True))
        mins.append(jnp.min(k, axis=1, keepdims=True))
    maxk = jnp.maximum(jnp.maximum(jnp.maximum(maxs[0], maxs[1]),
                                   jnp.maximum(maxs[2], maxs[3])),
                       jnp.maximum(jnp.maximum(maxs[4], maxs[5]),
                                   jnp.maximum(maxs[6], maxs[7])))
    mink = jnp.minimum(jnp.minimum(jnp.minimum(mins[0], mins[1]),
                                   jnp.minimum(mins[2], mins[3])),
                       jnp.minimum(jnp.minimum(mins[4], mins[5]),
                                   jnp.minimum(mins[6], mins[7])))
    win = (1 << 23) - 1
    t0h = maxk - win
    u0h = mink + win
    ct0, cu0 = counts(t0h, u0h)
    hint_ok = jnp.logical_and(jnp.all(ct0 >= _KMAX), jnp.all(cu0 >= _KMIN))

    def short_search(_):
        return jax.lax.fori_loop(0, 23, search_body(23), (t0h, u0h))

    def full_search(_):
        zero = jnp.zeros((rows, 1), jnp.int32)
        ct, cu = counts(zero, zero - 1)
        t0 = jnp.where(ct >= _KMAX, zero, zero + _INT_MIN)
        u0 = jnp.where(cu >= _KMIN, zero - 1, zero + _INT_MAX)
        return jax.lax.fori_loop(0, 31, search_body(31), (t0, u0))

    t, u = jax.lax.cond(hint_ok, short_search, full_search, None)

    cnt_gt, cnt_lt = counts(t + 1, u - 1)
    seg = n // 8
    gs, ls = [], []
    for sidx in range(8):
        ks = keys_ref[:, sidx * seg:(sidx + 1) * seg]
        xs = x_ref[:, sidx * seg:(sidx + 1) * seg]
        gs.append(jnp.sum(jnp.where(ks > t, xs, 0.0), axis=1, keepdims=True))
        ls.append(jnp.sum(jnp.where(ks < u, xs, 0.0), axis=1, keepdims=True))
    s_gt = sum(gs)
    s_lt = sum(ls)
    tval = jax.lax.bitcast_convert_type(
        jnp.where(t >= 0, t, t ^ _SIGN_MASK), jnp.float32)
    uval = jax.lax.bitcast_convert_type(
        jnp.where(u >= 0, u, u ^ _SIGN_MASK), jnp.float32)
    top = s_gt + (_KMAX - cnt_gt).astype(jnp.float32) * tval
    bot = s_lt + (_KMIN - cnt_lt).astype(jnp.float32) * uval
    res = top / _KMAX + bot / _KMIN
    o_ref[0] = jnp.broadcast_to(res, (rows, 128))


def kernel(inputs):
    b, h, w, c = inputs.shape
    n = h * w
    rows = b * c
    rg = 16
    assert rows % rg == 0 and n % 128 == 0
    g = rows // rg
    x = jnp.transpose(inputs, (0, 3, 1, 2)).reshape(rows, n)
    out = pl.pallas_call(
        _select_body,
        grid=(g,),
        in_specs=[pl.BlockSpec((rg, n), lambda i: (i, 0))],
        out_specs=pl.BlockSpec((1, rg, 128), lambda i: (i, 0, 0)),
        out_shape=jax.ShapeDtypeStruct((g, rg, 128), jnp.float32),
        scratch_shapes=[pltpu.VMEM((rg, n), jnp.int32)],
    )(x)
    return out[:, :, 0].reshape(b, c)
